# baseline (speedup 1.0000x reference)
import functools

import jax
import jax.numpy as jnp
from jax import lax
from jax.experimental import pallas as pl
from jax.experimental.pallas import tpu as pltpu
from jax.experimental.pallas import tpu_sc as plsc

N = 16384
F = 26
V = 100000
D = 64
S = 768
K = F * D

_NC = 2
_NS = 16
_NW = _NC * _NS
_C = 512
_NPAIR = F // 2


def _sc_concat_body(emb_hbm, *refs):
    outs = refs[:_NPAIR]
    buf0, buf1 = refs[_NPAIR], refs[_NPAIR + 1]
    wid = lax.axis_index("s") * _NC + lax.axis_index("c")
    n0 = wid * _C

    for p in range(_NPAIR):
        pltpu.sync_copy(emb_hbm.at[2 * p, pl.ds(n0, _C), :], buf0)
        pltpu.sync_copy(emb_hbm.at[2 * p + 1, pl.ds(n0, _C), :], buf1)
        pltpu.sync_copy(buf0, outs[p].at[pl.ds(n0, _C), pl.ds(0, D)])
        pltpu.sync_copy(buf1, outs[p].at[pl.ds(n0, _C), pl.ds(D, D)])


_sc_concat = functools.partial(
    pl.kernel,
    out_type=tuple(
        jax.ShapeDtypeStruct((N, 2 * D), jnp.float32) for _ in range(_NPAIR)
    ),
    mesh=plsc.VectorSubcoreMesh(core_axis_name="c", subcore_axis_name="s"),
    compiler_params=pltpu.CompilerParams(use_tc_tiling_on_sc=False),
    scratch_types=[
        pltpu.VMEM((_C, D), jnp.float32),
        pltpu.VMEM((_C, D), jnp.float32),
    ],
)(_sc_concat_body)


_BN = 512


def _mm_body(*refs):
    g_refs = refs[:_NPAIR]
    s_ref, w_ref, b_ref, o_ref = refs[_NPAIR:]
    g = jnp.concatenate([r[...] for r in g_refs], axis=1)
    acc = lax.dot_general(
        g, w_ref[...],
        (((1,), (1,)), ((), ())),
        preferred_element_type=jnp.float32,
    )
    o_ref[:, :S] = s_ref[...]
    o_ref[:, S:] = acc + b_ref[...]


def kernel(sentence_embeddings, categorical_data, tables, W, b):
    emb = jax.vmap(lambda t, i: jnp.take(t, i, axis=0))(
        tables, categorical_data)
    pair_blocks = _sc_concat(emb)
    out = pl.pallas_call(
        _mm_body,
        grid=(N // _BN,),
        in_specs=[
            pl.BlockSpec((_BN, 2 * D), lambda i: (i, 0))
            for _ in range(_NPAIR)
        ] + [
            pl.BlockSpec((_BN, S), lambda i: (i, 0)),
            pl.BlockSpec((S, K), lambda i: (0, 0)),
            pl.BlockSpec((1, S), lambda i: (0, 0)),
        ],
        out_specs=pl.BlockSpec((_BN, 2 * S), lambda i: (i, 0)),
        out_shape=jax.ShapeDtypeStruct((N, 2 * S), jnp.float32),
    )(*pair_blocks, sentence_embeddings, W, b.reshape(1, S))
    return out


# speedup vs baseline: 1.2572x; 1.2572x over previous
import jax
import jax.numpy as jnp
from jax import lax
from jax.experimental import pallas as pl

N = 16384
F = 26
V = 100000
D = 64
S = 768
K = F * D

_BN = 512


def _mm_body(e_ref, s_ref, w_ref, b_ref, o_ref):
    g = jnp.concatenate([e_ref[f] for f in range(F)], axis=1)
    acc = lax.dot_general(
        g, w_ref[...],
        (((1,), (1,)), ((), ())),
        preferred_element_type=jnp.float32,
    )
    o_ref[:, :S] = s_ref[...]
    o_ref[:, S:] = acc + b_ref[...]


def kernel(sentence_embeddings, categorical_data, tables, W, b):
    emb = jax.vmap(lambda t, i: jnp.take(t, i, axis=0))(
        tables, categorical_data)
    out = pl.pallas_call(
        _mm_body,
        grid=(N // _BN,),
        in_specs=[
            pl.BlockSpec((F, _BN, D), lambda i: (0, i, 0)),
            pl.BlockSpec((_BN, S), lambda i: (i, 0)),
            pl.BlockSpec((S, K), lambda i: (0, 0)),
            pl.BlockSpec((1, S), lambda i: (0, 0)),
        ],
        out_specs=pl.BlockSpec((_BN, 2 * S), lambda i: (i, 0)),
        out_shape=jax.ShapeDtypeStruct((N, 2 * S), jnp.float32),
    )(emb, sentence_embeddings, W, b.reshape(1, S))
    return out
